# baseline (speedup 1.0000x reference)
import functools

import jax
import jax.numpy as jnp
from jax import lax
from jax.experimental import pallas as pl
from jax.experimental.pallas import tpu as pltpu
from jax.experimental.pallas import tpu_sc as plsc

E = 8
TOPK = 2
IN = 1024
RH = 512
RH2 = 256
H1 = 1024
H2 = 512
NC = 256
N = 2048
NB = 256
EP = 128
NT = N // NB
NBR = 512
NTR = N // NBR
B = 128
NCH = (N * TOPK) // B + E - 1
NCHG = NCH + 1
CHA = 48
APAD = NCHG * B
NWORK = 32
RPT = APAD // NWORK
GB = 48
TPT = N // NWORK


def _router_body(x_ref, wr1_ref, br1_ref, wr2_ref, br2_ref, wr3_ref, br3_ref,
                 probs_ref, meta_ref, metat_ref, runc_ref):
    t = pl.program_id(0)
    x = x_ref[...]
    h = jnp.maximum(
        jnp.dot(x, wr1_ref[...], preferred_element_type=jnp.float32)
        + br1_ref[...], 0.0)
    h = jnp.maximum(
        jnp.dot(h, wr2_ref[...], preferred_element_type=jnp.float32)
        + br2_ref[...], 0.0)
    s = jnp.dot(h, wr3_ref[...], preferred_element_type=jnp.float32) \
        + br3_ref[...]
    lane = lax.broadcasted_iota(jnp.int32, (NBR, EP), 1)
    s = jnp.where(lane < E, s, -1e30)
    m = jnp.max(s, axis=1, keepdims=True)
    p = jnp.exp(s - m)
    probs = p / jnp.sum(p, axis=1, keepdims=True)
    probs_ref[...] = probs
    v1 = jnp.max(probs, axis=1, keepdims=True)
    i1 = jnp.min(jnp.where(probs == v1, lane, EP), axis=1, keepdims=True)
    pm = jnp.where(lane == i1, -1.0, probs)
    v2 = jnp.max(pm, axis=1, keepdims=True)
    i2 = jnp.min(jnp.where(pm == v2, lane, EP), axis=1, keepdims=True)

    m0 = (lane == i1).astype(jnp.float32)
    m1 = (lane == i2).astype(jnp.float32)
    msum = m0 + m1
    ri = lax.broadcasted_iota(jnp.int32, (NBR, NBR), 0)
    ci = lax.broadcasted_iota(jnp.int32, (NBR, NBR), 1)
    tri = (ri > ci).astype(jnp.float32)
    within = jnp.dot(tri, msum, preferred_element_type=jnp.float32)

    @pl.when(t == 0)
    def _init():
        runc_ref[...] = jnp.zeros((1, EP), jnp.float32)

    runb = runc_ref[...]
    base = runb + within
    r0 = jnp.sum(jnp.where(lane == i1, base, 0.0), axis=1, keepdims=True)
    r1 = jnp.sum(jnp.where(lane == i2, base, 0.0), axis=1, keepdims=True)
    runc_ref[...] = runb + jnp.sum(msum, axis=0, keepdims=True)

    meta = (jnp.where(lane == 0, v1 * 0.5, 0.0)
            + jnp.where(lane == 1, v2 * 0.5, 0.0))
    meta_ref[...] = meta

    meta8 = jnp.concatenate(
        [i1.astype(jnp.float32), i2.astype(jnp.float32), r0, r1,
         jnp.zeros((NBR, 4), jnp.float32)], axis=1)
    mt = jnp.transpose(meta8)
    cntrow = jnp.concatenate(
        [runc_ref[...], jnp.zeros((1, NBR - EP), jnp.float32)], axis=1)
    subl = lax.broadcasted_iota(jnp.int32, (8, NBR), 0)
    metat_ref[...] = jnp.where(subl == 6, cntrow, mt)


def _pos_body(mtt_ref, mtc_ref, post_ref):
    blk = mtt_ref[...]
    cblk = mtc_ref[...]
    subl = lax.broadcasted_iota(jnp.int32, (8, NB), 0)
    lane = lax.broadcasted_iota(jnp.int32, (8, NB), 1)
    starts = []
    bounds = []
    run = 0.0
    for e in range(E):
        cnt_e = jnp.sum(jnp.where((subl == 6) & (lane == e), cblk, 0.0))
        cc_e = jnp.floor((cnt_e + (B - 1)) * (1.0 / B))
        starts.append(run * B)
        run = run + cc_e
        bounds.append(run)
    i1row = blk[0:1, :]
    i2row = blk[1:2, :]
    pos0 = blk[2:3, :]
    pos1 = blk[3:4, :]
    for e in range(E):
        pos0 = pos0 + jnp.where(i1row == e, starts[e], 0.0)
        pos1 = pos1 + jnp.where(i2row == e, starts[e], 0.0)
    jrow = lane[0:1, :].astype(jnp.float32)
    acc = jnp.zeros((1, NB), jnp.float32)
    for e in range(E):
        acc = acc + (jrow >= bounds[e]).astype(jnp.float32)
    ce_row = jnp.minimum(acc, float(E - 1))
    cv_row = (acc < E).astype(jnp.float32)
    post_ref[...] = (jnp.where(subl == 0, pos0, 0.0)
                     + jnp.where(subl == 1, pos1, 0.0)
                     + jnp.where(subl == 4, ce_row, 0.0)
                     + jnp.where(subl == 5, cv_row, 0.0))


def _scatter_sc_body(post_hbm, x_hbm, xs_hbm, p0i_hbm, p1i_hbm,
                     pv, idx0v, idx1v, xbuf, s0, s1):
    c = lax.axis_index("c")
    s = lax.axis_index("s")
    w = s * 2 + c
    seg = w // 2
    half = w % 2
    dx = pltpu.async_copy(x_hbm.at[pl.ds(TPT * w, TPT)], xbuf, s0)
    pltpu.sync_copy(post_hbm.at[:, pl.ds(seg * 128, 128)], pv)
    for j in range(TPT // 16):
        idx0v[pl.ds(16 * j, 16)] = \
            pv[0, pl.ds(half * TPT + 16 * j, 16)].astype(jnp.int32)
        idx1v[pl.ds(16 * j, 16)] = \
            pv[1, pl.ds(half * TPT + 16 * j, 16)].astype(jnp.int32)
    dp0 = pltpu.async_copy(idx0v, p0i_hbm.at[pl.ds(TPT * w, TPT)], s1)
    dx.wait()
    d0 = pltpu.async_copy(xbuf, xs_hbm.at[idx0v], s0)
    dp0.wait()
    dp1 = pltpu.async_copy(idx1v, p1i_hbm.at[pl.ds(TPT * w, TPT)], s1)
    d0.wait()
    d1 = pltpu.async_copy(xbuf, xs_hbm.at[idx1v], s0)
    dp1.wait()
    d1.wait()


def _gout_sc_body(p0_hbm, p1_hbm, ys_hbm, g0_hbm, g1_hbm,
                  p0v, p1v, buf0, buf1, s0, s1, w0, w1):
    c = lax.axis_index("c")
    s = lax.axis_index("s")
    wid = s * 2 + c
    base = wid * TPT
    pltpu.sync_copy(p0_hbm.at[pl.ds(base, TPT)], p0v)
    pltpu.sync_copy(p1_hbm.at[pl.ds(base, TPT)], p1v)
    d0 = pltpu.async_copy(ys_hbm.at[p0v], buf0, s0)
    d1 = pltpu.async_copy(ys_hbm.at[p1v], buf1, s1)
    d0.wait()
    wb0 = pltpu.async_copy(buf0, g0_hbm.at[pl.ds(base, TPT)], w0)
    d1.wait()
    wb1 = pltpu.async_copy(buf1, g1_hbm.at[pl.ds(base, TPT)], w1)
    wb0.wait()
    wb1.wait()


def _experts_body(ce_ref, cv_ref, xs_ref, we1_ref, be1_ref, we2_ref, be2_ref,
                  we3_ref, be3_ref, ys_ref):
    c = pl.program_id(0)

    @pl.when(cv_ref[c] == 1)
    def _compute():
        xb = xs_ref[...]
        h1 = jnp.maximum(
            jnp.dot(xb, we1_ref[0], preferred_element_type=jnp.float32)
            + be1_ref[0], 0.0)
        h2 = jnp.maximum(
            jnp.dot(h1, we2_ref[0], preferred_element_type=jnp.float32)
            + be2_ref[0], 0.0)
        ys_ref[...] = jnp.dot(
            h2, we3_ref[0], preferred_element_type=jnp.float32) + be3_ref[0]


def _combine_body(meta_ref, g0_ref, g1_ref, out_ref):
    lane = lax.broadcasted_iota(jnp.int32, (NB, EP), 1)
    meta = meta_ref[...]
    wa = jnp.sum(jnp.where(lane == 0, meta, 0.0), axis=1, keepdims=True)
    wb = jnp.sum(jnp.where(lane == 1, meta, 0.0), axis=1, keepdims=True)
    out_ref[...] = wa * g0_ref[...] + wb * g1_ref[...]


@jax.jit
def kernel(x, Wr1, br1, Wr2, br2, Wr3, br3, We1, be1, We2, be2, We3, be3):
    wr3p = jnp.pad(Wr3, ((0, 0), (0, EP - E)))
    br3p = jnp.pad(br3, (0, EP - E)).reshape(1, EP)

    probs_full, meta, meta_t = pl.pallas_call(
        _router_body,
        grid=(NTR,),
        in_specs=[
            pl.BlockSpec((NBR, IN), lambda t: (t, 0)),
            pl.BlockSpec((IN, RH), lambda t: (0, 0)),
            pl.BlockSpec((1, RH), lambda t: (0, 0)),
            pl.BlockSpec((RH, RH2), lambda t: (0, 0)),
            pl.BlockSpec((1, RH2), lambda t: (0, 0)),
            pl.BlockSpec((RH2, EP), lambda t: (0, 0)),
            pl.BlockSpec((1, EP), lambda t: (0, 0)),
        ],
        out_specs=[
            pl.BlockSpec((NBR, EP), lambda t: (t, 0)),
            pl.BlockSpec((NBR, EP), lambda t: (t, 0)),
            pl.BlockSpec((8, NBR), lambda t: (0, t)),
        ],
        out_shape=[
            jax.ShapeDtypeStruct((N, EP), jnp.float32),
            jax.ShapeDtypeStruct((N, EP), jnp.float32),
            jax.ShapeDtypeStruct((8, N), jnp.float32),
        ],
        scratch_shapes=[pltpu.VMEM((1, EP), jnp.float32)],
        compiler_params=pltpu.CompilerParams(
            dimension_semantics=("arbitrary",)),
    )(x, Wr1, br1.reshape(1, RH), Wr2, br2.reshape(1, RH2), wr3p, br3p)

    mesh = plsc.VectorSubcoreMesh(core_axis_name="c", subcore_axis_name="s")

    post = pl.pallas_call(
        _pos_body,
        grid=(NT,),
        in_specs=[
            pl.BlockSpec((8, NB), lambda t: (0, t)),
            pl.BlockSpec((8, NB), lambda t: (0, ((NTR - 1) * NBR) // NB)),
        ],
        out_specs=pl.BlockSpec((8, NB), lambda t: (0, t)),
        out_shape=jax.ShapeDtypeStruct((8, N), jnp.float32),
    )(meta_t, meta_t)

    chunk_expert = post[4, :CHA].astype(jnp.int32)
    chunk_valid = post[5, :CHA].astype(jnp.int32)

    scatter_sc = functools.partial(
        pl.kernel,
        out_type=[
            jax.ShapeDtypeStruct((APAD, IN), jnp.float32),
            jax.ShapeDtypeStruct((N,), jnp.int32),
            jax.ShapeDtypeStruct((N,), jnp.int32),
        ],
        mesh=mesh,
        compiler_params=pltpu.CompilerParams(needs_layout_passes=False),
        scratch_types=[
            pltpu.VMEM((8, 128), jnp.float32),
            pltpu.VMEM((TPT,), jnp.int32),
            pltpu.VMEM((TPT,), jnp.int32),
            pltpu.VMEM((TPT, IN), jnp.float32),
            pltpu.SemaphoreType.DMA,
            pltpu.SemaphoreType.DMA,
        ],
    )(_scatter_sc_body)
    xs, pos0, pos1 = scatter_sc(post, x)

    ys = pl.pallas_call(
        _experts_body,
        grid_spec=pltpu.PrefetchScalarGridSpec(
            num_scalar_prefetch=2,
            grid=(NCHG,),
            in_specs=[
                pl.BlockSpec((B, IN), lambda c, ce, cv: (c, 0)),
                pl.BlockSpec((1, IN, H1), lambda c, ce, cv: (ce[c], 0, 0)),
                pl.BlockSpec((1, 1, H1), lambda c, ce, cv: (ce[c], 0, 0)),
                pl.BlockSpec((1, H1, H2), lambda c, ce, cv: (ce[c], 0, 0)),
                pl.BlockSpec((1, 1, H2), lambda c, ce, cv: (ce[c], 0, 0)),
                pl.BlockSpec((1, H2, NC), lambda c, ce, cv: (ce[c], 0, 0)),
                pl.BlockSpec((1, 1, NC), lambda c, ce, cv: (ce[c], 0, 0)),
            ],
            out_specs=pl.BlockSpec((B, NC), lambda c, ce, cv: (c, 0)),
        ),
        out_shape=jax.ShapeDtypeStruct((APAD, NC), jnp.float32),
        compiler_params=pltpu.CompilerParams(
            dimension_semantics=("arbitrary",)),
    )(chunk_expert, chunk_valid, xs, We1, be1.reshape(E, 1, H1),
      We2, be2.reshape(E, 1, H2), We3, be3.reshape(E, 1, NC))

    gout_sc = functools.partial(
        pl.kernel,
        out_type=[
            jax.ShapeDtypeStruct((N, NC), jnp.float32),
            jax.ShapeDtypeStruct((N, NC), jnp.float32),
        ],
        mesh=mesh,
        compiler_params=pltpu.CompilerParams(needs_layout_passes=False),
        scratch_types=[
            pltpu.VMEM((TPT,), jnp.int32),
            pltpu.VMEM((TPT,), jnp.int32),
            pltpu.VMEM((TPT, NC), jnp.float32),
            pltpu.VMEM((TPT, NC), jnp.float32),
            pltpu.SemaphoreType.DMA,
            pltpu.SemaphoreType.DMA,
            pltpu.SemaphoreType.DMA,
            pltpu.SemaphoreType.DMA,
        ],
    )(_gout_sc_body)
    g0, g1 = gout_sc(pos0, pos1, ys)

    out = pl.pallas_call(
        _combine_body,
        grid=(NT,),
        in_specs=[
            pl.BlockSpec((NB, EP), lambda t: (t, 0)),
            pl.BlockSpec((NB, NC), lambda t: (t, 0)),
            pl.BlockSpec((NB, NC), lambda t: (t, 0)),
        ],
        out_specs=pl.BlockSpec((NB, NC), lambda t: (t, 0)),
        out_shape=jax.ShapeDtypeStruct((N, NC), jnp.float32),
    )(meta, g0, g1)

    return out, probs_full[:, :E]


# speedup vs baseline: 1.1478x; 1.1478x over previous
import functools

import jax
import jax.numpy as jnp
from jax import lax
from jax.experimental import pallas as pl
from jax.experimental.pallas import tpu as pltpu
from jax.experimental.pallas import tpu_sc as plsc

E = 8
TOPK = 2
IN = 1024
RH = 512
RH2 = 256
H1 = 1024
H2 = 512
NC = 256
N = 2048
NB = 256
EP = 128
NT = N // NB
NBR = 512
NTR = N // NBR
B = 256
NCH = (N * TOPK) // B + E - 1
NCHG = NCH + 1
CHA = 32
APAD = NCHG * B
NWORK = 32
RPT = APAD // NWORK
GB = 48
TPT = N // NWORK


def _router_body(x_ref, wr1_ref, br1_ref, wr2_ref, br2_ref, wr3_ref, br3_ref,
                 probs_ref, meta_ref, post_ref, runc_ref, mts_ref):
    t = pl.program_id(0)

    @pl.when(t < NTR)
    def _route():
        x = x_ref[...]
        h = jnp.maximum(
            jnp.dot(x, wr1_ref[...], preferred_element_type=jnp.float32)
            + br1_ref[...], 0.0)
        h = jnp.maximum(
            jnp.dot(h, wr2_ref[...], preferred_element_type=jnp.float32)
            + br2_ref[...], 0.0)
        s = jnp.dot(h, wr3_ref[...], preferred_element_type=jnp.float32) \
            + br3_ref[...]
        lane = lax.broadcasted_iota(jnp.int32, (NBR, EP), 1)
        s = jnp.where(lane < E, s, -1e30)
        m = jnp.max(s, axis=1, keepdims=True)
        p = jnp.exp(s - m)
        probs = p / jnp.sum(p, axis=1, keepdims=True)
        probs_ref[...] = probs
        v1 = jnp.max(probs, axis=1, keepdims=True)
        i1 = jnp.min(jnp.where(probs == v1, lane, EP), axis=1, keepdims=True)
        pm = jnp.where(lane == i1, -1.0, probs)
        v2 = jnp.max(pm, axis=1, keepdims=True)
        i2 = jnp.min(jnp.where(pm == v2, lane, EP), axis=1, keepdims=True)

        m0 = (lane == i1).astype(jnp.float32)
        m1 = (lane == i2).astype(jnp.float32)
        msum = m0 + m1
        ri = lax.broadcasted_iota(jnp.int32, (NBR, NBR), 0)
        ci = lax.broadcasted_iota(jnp.int32, (NBR, NBR), 1)
        tri = (ri > ci).astype(jnp.float32)
        within = jnp.dot(tri, msum, preferred_element_type=jnp.float32)

        @pl.when(t == 0)
        def _init():
            runc_ref[...] = jnp.zeros((1, EP), jnp.float32)

        runb = runc_ref[...]
        base = runb + within
        r0 = jnp.sum(jnp.where(lane == i1, base, 0.0), axis=1, keepdims=True)
        r1 = jnp.sum(jnp.where(lane == i2, base, 0.0), axis=1, keepdims=True)
        runc_ref[...] = runb + jnp.sum(msum, axis=0, keepdims=True)

        meta_ref[...] = (jnp.where(lane == 0, v1 * 0.5, 0.0)
                         + jnp.where(lane == 1, v2 * 0.5, 0.0))

        meta8 = jnp.concatenate(
            [i1.astype(jnp.float32), i2.astype(jnp.float32), r0, r1,
             jnp.zeros((NBR, 4), jnp.float32)], axis=1)
        mts_ref[:, pl.ds(t * NBR, NBR)] = jnp.transpose(meta8)

    @pl.when(t == NTR)
    def _positions():
        runc = runc_ref[...]
        elane = lax.broadcasted_iota(jnp.int32, (1, EP), 1)
        subl = lax.broadcasted_iota(jnp.int32, (8, N), 0)
        starts = []
        bounds = []
        run = 0.0
        for e in range(E):
            cnt_e = jnp.sum(jnp.where(elane == e, runc, 0.0))
            cc_e = jnp.floor((cnt_e + (B - 1)) * (1.0 / B))
            starts.append(run * B)
            run = run + cc_e
            bounds.append(run)
        blk = mts_ref[...]
        i1row = blk[0:1, :]
        i2row = blk[1:2, :]
        pos0 = blk[2:3, :]
        pos1 = blk[3:4, :]
        for e in range(E):
            pos0 = pos0 + jnp.where(i1row == e, starts[e], 0.0)
            pos1 = pos1 + jnp.where(i2row == e, starts[e], 0.0)
        jrow = lax.broadcasted_iota(jnp.int32, (1, N), 1).astype(jnp.float32)
        acc = jnp.zeros((1, N), jnp.float32)
        for e in range(E):
            acc = acc + (jrow >= bounds[e]).astype(jnp.float32)
        ce_row = jnp.minimum(acc, float(E - 1))
        cv_row = (acc < E).astype(jnp.float32)
        post_ref[...] = (jnp.where(subl == 0, pos0, 0.0)
                         + jnp.where(subl == 1, pos1, 0.0)
                         + jnp.where(subl == 4, ce_row, 0.0)
                         + jnp.where(subl == 5, cv_row, 0.0))


def _scatter_sc_body(post_hbm, x_hbm, xs_hbm, p0i_hbm, p1i_hbm,
                     pv, idx0v, idx1v, xbuf, s0, s1):
    c = lax.axis_index("c")
    s = lax.axis_index("s")
    w = s * 2 + c
    seg = w // 2
    half = w % 2
    dx = pltpu.async_copy(x_hbm.at[pl.ds(TPT * w, TPT)], xbuf, s0)
    pltpu.sync_copy(post_hbm.at[:, pl.ds(seg * 128, 128)], pv)
    for j in range(TPT // 16):
        idx0v[pl.ds(16 * j, 16)] = \
            pv[0, pl.ds(half * TPT + 16 * j, 16)].astype(jnp.int32)
        idx1v[pl.ds(16 * j, 16)] = \
            pv[1, pl.ds(half * TPT + 16 * j, 16)].astype(jnp.int32)
    dp0 = pltpu.async_copy(idx0v, p0i_hbm.at[pl.ds(TPT * w, TPT)], s1)
    dx.wait()
    d0 = pltpu.async_copy(xbuf, xs_hbm.at[idx0v], s0)
    dp0.wait()
    dp1 = pltpu.async_copy(idx1v, p1i_hbm.at[pl.ds(TPT * w, TPT)], s1)
    d0.wait()
    d1 = pltpu.async_copy(xbuf, xs_hbm.at[idx1v], s0)
    dp1.wait()
    d1.wait()


def _gout_sc_body(p0_hbm, p1_hbm, ys_hbm, g0_hbm, g1_hbm,
                  p0v, p1v, buf0, buf1, s0, s1, w0, w1):
    c = lax.axis_index("c")
    s = lax.axis_index("s")
    wid = s * 2 + c
    base = wid * TPT
    pltpu.sync_copy(p0_hbm.at[pl.ds(base, TPT)], p0v)
    pltpu.sync_copy(p1_hbm.at[pl.ds(base, TPT)], p1v)
    d0 = pltpu.async_copy(ys_hbm.at[p0v], buf0, s0)
    d1 = pltpu.async_copy(ys_hbm.at[p1v], buf1, s1)
    d0.wait()
    wb0 = pltpu.async_copy(buf0, g0_hbm.at[pl.ds(base, TPT)], w0)
    d1.wait()
    wb1 = pltpu.async_copy(buf1, g1_hbm.at[pl.ds(base, TPT)], w1)
    wb0.wait()
    wb1.wait()


def _experts_body(ce_ref, cv_ref, xs_ref, we1_ref, be1_ref, we2_ref, be2_ref,
                  we3_ref, be3_ref, ys_ref):
    c = pl.program_id(0)

    @pl.when(cv_ref[c] == 1)
    def _compute():
        xb = xs_ref[...]
        h1 = jnp.maximum(
            jnp.dot(xb, we1_ref[0], preferred_element_type=jnp.float32)
            + be1_ref[0], 0.0)
        h2 = jnp.maximum(
            jnp.dot(h1, we2_ref[0], preferred_element_type=jnp.float32)
            + be2_ref[0], 0.0)
        ys_ref[...] = jnp.dot(
            h2, we3_ref[0], preferred_element_type=jnp.float32) + be3_ref[0]


def _combine_body(meta_ref, g0_ref, g1_ref, out_ref):
    lane = lax.broadcasted_iota(jnp.int32, (NB, EP), 1)
    meta = meta_ref[...]
    wa = jnp.sum(jnp.where(lane == 0, meta, 0.0), axis=1, keepdims=True)
    wb = jnp.sum(jnp.where(lane == 1, meta, 0.0), axis=1, keepdims=True)
    out_ref[...] = wa * g0_ref[...] + wb * g1_ref[...]


@jax.jit
def kernel(x, Wr1, br1, Wr2, br2, Wr3, br3, We1, be1, We2, be2, We3, be3):
    wr3p = jnp.pad(Wr3, ((0, 0), (0, EP - E)))
    br3p = jnp.pad(br3, (0, EP - E)).reshape(1, EP)

    probs_full, meta, post = pl.pallas_call(
        _router_body,
        grid=(NTR + 1,),
        in_specs=[
            pl.BlockSpec((NBR, IN), lambda t: (jnp.minimum(t, NTR - 1), 0)),
            pl.BlockSpec((IN, RH), lambda t: (0, 0)),
            pl.BlockSpec((1, RH), lambda t: (0, 0)),
            pl.BlockSpec((RH, RH2), lambda t: (0, 0)),
            pl.BlockSpec((1, RH2), lambda t: (0, 0)),
            pl.BlockSpec((RH2, EP), lambda t: (0, 0)),
            pl.BlockSpec((1, EP), lambda t: (0, 0)),
        ],
        out_specs=[
            pl.BlockSpec((NBR, EP), lambda t: (jnp.minimum(t, NTR - 1), 0)),
            pl.BlockSpec((NBR, EP), lambda t: (jnp.minimum(t, NTR - 1), 0)),
            pl.BlockSpec((8, N), lambda t: (0, 0)),
        ],
        out_shape=[
            jax.ShapeDtypeStruct((N, EP), jnp.float32),
            jax.ShapeDtypeStruct((N, EP), jnp.float32),
            jax.ShapeDtypeStruct((8, N), jnp.float32),
        ],
        scratch_shapes=[pltpu.VMEM((1, EP), jnp.float32),
                        pltpu.VMEM((8, N), jnp.float32)],
        compiler_params=pltpu.CompilerParams(
            dimension_semantics=("arbitrary",)),
    )(x, Wr1, br1.reshape(1, RH), Wr2, br2.reshape(1, RH2), wr3p, br3p)

    mesh = plsc.VectorSubcoreMesh(core_axis_name="c", subcore_axis_name="s")

    chunk_expert = post[4, :CHA].astype(jnp.int32)
    chunk_valid = post[5, :CHA].astype(jnp.int32)

    scatter_sc = functools.partial(
        pl.kernel,
        out_type=[
            jax.ShapeDtypeStruct((APAD, IN), jnp.float32),
            jax.ShapeDtypeStruct((N,), jnp.int32),
            jax.ShapeDtypeStruct((N,), jnp.int32),
        ],
        mesh=mesh,
        compiler_params=pltpu.CompilerParams(needs_layout_passes=False),
        scratch_types=[
            pltpu.VMEM((8, 128), jnp.float32),
            pltpu.VMEM((TPT,), jnp.int32),
            pltpu.VMEM((TPT,), jnp.int32),
            pltpu.VMEM((TPT, IN), jnp.float32),
            pltpu.SemaphoreType.DMA,
            pltpu.SemaphoreType.DMA,
        ],
    )(_scatter_sc_body)
    xs, pos0, pos1 = scatter_sc(post, x)

    ys = pl.pallas_call(
        _experts_body,
        grid_spec=pltpu.PrefetchScalarGridSpec(
            num_scalar_prefetch=2,
            grid=(NCHG,),
            in_specs=[
                pl.BlockSpec((B, IN), lambda c, ce, cv: (c, 0)),
                pl.BlockSpec((1, IN, H1), lambda c, ce, cv: (ce[c], 0, 0)),
                pl.BlockSpec((1, 1, H1), lambda c, ce, cv: (ce[c], 0, 0)),
                pl.BlockSpec((1, H1, H2), lambda c, ce, cv: (ce[c], 0, 0)),
                pl.BlockSpec((1, 1, H2), lambda c, ce, cv: (ce[c], 0, 0)),
                pl.BlockSpec((1, H2, NC), lambda c, ce, cv: (ce[c], 0, 0)),
                pl.BlockSpec((1, 1, NC), lambda c, ce, cv: (ce[c], 0, 0)),
            ],
            out_specs=pl.BlockSpec((B, NC), lambda c, ce, cv: (c, 0)),
        ),
        out_shape=jax.ShapeDtypeStruct((APAD, NC), jnp.float32),
        compiler_params=pltpu.CompilerParams(
            dimension_semantics=("arbitrary",)),
    )(chunk_expert, chunk_valid, xs, We1, be1.reshape(E, 1, H1),
      We2, be2.reshape(E, 1, H2), We3, be3.reshape(E, 1, NC))

    gout_sc = functools.partial(
        pl.kernel,
        out_type=[
            jax.ShapeDtypeStruct((N, NC), jnp.float32),
            jax.ShapeDtypeStruct((N, NC), jnp.float32),
        ],
        mesh=mesh,
        compiler_params=pltpu.CompilerParams(needs_layout_passes=False),
        scratch_types=[
            pltpu.VMEM((TPT,), jnp.int32),
            pltpu.VMEM((TPT,), jnp.int32),
            pltpu.VMEM((TPT, NC), jnp.float32),
            pltpu.VMEM((TPT, NC), jnp.float32),
            pltpu.SemaphoreType.DMA,
            pltpu.SemaphoreType.DMA,
            pltpu.SemaphoreType.DMA,
            pltpu.SemaphoreType.DMA,
        ],
    )(_gout_sc_body)
    g0, g1 = gout_sc(pos0, pos1, ys)

    out = pl.pallas_call(
        _combine_body,
        grid=(NT,),
        in_specs=[
            pl.BlockSpec((NB, EP), lambda t: (t, 0)),
            pl.BlockSpec((NB, NC), lambda t: (t, 0)),
            pl.BlockSpec((NB, NC), lambda t: (t, 0)),
        ],
        out_specs=pl.BlockSpec((NB, NC), lambda t: (t, 0)),
        out_shape=jax.ShapeDtypeStruct((N, NC), jnp.float32),
    )(meta, g0, g1)

    return out, probs_full[:, :E]


# speedup vs baseline: 1.1546x; 1.0059x over previous
import functools

import jax
import jax.numpy as jnp
from jax import lax
from jax.experimental import pallas as pl
from jax.experimental.pallas import tpu as pltpu
from jax.experimental.pallas import tpu_sc as plsc

E = 8
TOPK = 2
IN = 1024
RH = 512
RH2 = 256
H1 = 1024
H2 = 512
NC = 256
N = 2048
NB = 256
EP = 128
NT = N // NB
NBR = 1024
NTR = N // NBR
B = 256
NCH = (N * TOPK) // B + E - 1
NCHG = NCH + 1
CHA = 32
APAD = NCHG * B
NWORK = 32
RPT = APAD // NWORK
GB = 48
TPT = N // NWORK


def _router_body(x_ref, wr1_ref, br1_ref, wr2_ref, br2_ref, wr3_ref, br3_ref,
                 probs_ref, meta_ref, post_ref, runc_ref, mts_ref):
    t = pl.program_id(0)

    @pl.when(t < NTR)
    def _route():
        x = x_ref[...]
        h = jnp.maximum(
            jnp.dot(x, wr1_ref[...], preferred_element_type=jnp.float32)
            + br1_ref[...], 0.0)
        h = jnp.maximum(
            jnp.dot(h, wr2_ref[...], preferred_element_type=jnp.float32)
            + br2_ref[...], 0.0)
        s = jnp.dot(h, wr3_ref[...], preferred_element_type=jnp.float32) \
            + br3_ref[...]
        lane = lax.broadcasted_iota(jnp.int32, (NBR, EP), 1)
        s = jnp.where(lane < E, s, -1e30)
        m = jnp.max(s, axis=1, keepdims=True)
        p = jnp.exp(s - m)
        probs = p / jnp.sum(p, axis=1, keepdims=True)
        probs_ref[...] = probs
        v1 = jnp.max(probs, axis=1, keepdims=True)
        i1 = jnp.min(jnp.where(probs == v1, lane, EP), axis=1, keepdims=True)
        pm = jnp.where(lane == i1, -1.0, probs)
        v2 = jnp.max(pm, axis=1, keepdims=True)
        i2 = jnp.min(jnp.where(pm == v2, lane, EP), axis=1, keepdims=True)

        m0 = (lane == i1).astype(jnp.float32)
        m1 = (lane == i2).astype(jnp.float32)
        msum = m0 + m1
        ri = lax.broadcasted_iota(jnp.int32, (NBR, NBR), 0)
        ci = lax.broadcasted_iota(jnp.int32, (NBR, NBR), 1)
        tri = (ri > ci).astype(jnp.float32)
        within = jnp.dot(tri, msum, preferred_element_type=jnp.float32)

        @pl.when(t == 0)
        def _init():
            runc_ref[...] = jnp.zeros((1, EP), jnp.float32)

        runb = runc_ref[...]
        base = runb + within
        r0 = jnp.sum(jnp.where(lane == i1, base, 0.0), axis=1, keepdims=True)
        r1 = jnp.sum(jnp.where(lane == i2, base, 0.0), axis=1, keepdims=True)
        runc_ref[...] = runb + jnp.sum(msum, axis=0, keepdims=True)

        meta_ref[...] = (jnp.where(lane == 0, v1 * 0.5, 0.0)
                         + jnp.where(lane == 1, v2 * 0.5, 0.0))

        meta8 = jnp.concatenate(
            [i1.astype(jnp.float32), i2.astype(jnp.float32), r0, r1,
             jnp.zeros((NBR, 4), jnp.float32)], axis=1)
        mts_ref[:, pl.ds(t * NBR, NBR)] = jnp.transpose(meta8)

    @pl.when(t == NTR)
    def _positions():
        runc = runc_ref[...]
        elane = lax.broadcasted_iota(jnp.int32, (1, EP), 1)
        subl = lax.broadcasted_iota(jnp.int32, (8, N), 0)
        starts = []
        bounds = []
        run = 0.0
        for e in range(E):
            cnt_e = jnp.sum(jnp.where(elane == e, runc, 0.0))
            cc_e = jnp.floor((cnt_e + (B - 1)) * (1.0 / B))
            starts.append(run * B)
            run = run + cc_e
            bounds.append(run)
        blk = mts_ref[...]
        i1row = blk[0:1, :]
        i2row = blk[1:2, :]
        pos0 = blk[2:3, :]
        pos1 = blk[3:4, :]
        for e in range(E):
            pos0 = pos0 + jnp.where(i1row == e, starts[e], 0.0)
            pos1 = pos1 + jnp.where(i2row == e, starts[e], 0.0)
        jrow = lax.broadcasted_iota(jnp.int32, (1, N), 1).astype(jnp.float32)
        acc = jnp.zeros((1, N), jnp.float32)
        for e in range(E):
            acc = acc + (jrow >= bounds[e]).astype(jnp.float32)
        ce_row = jnp.minimum(acc, float(E - 1))
        cv_row = (acc < E).astype(jnp.float32)
        post_ref[...] = (jnp.where(subl == 0, pos0, 0.0)
                         + jnp.where(subl == 1, pos1, 0.0)
                         + jnp.where(subl == 4, ce_row, 0.0)
                         + jnp.where(subl == 5, cv_row, 0.0))


def _scatter_sc_body(post_hbm, x_hbm, xs_hbm, p0i_hbm, p1i_hbm,
                     pv, idx0v, idx1v, xbuf, s0, s1):
    c = lax.axis_index("c")
    s = lax.axis_index("s")
    w = s * 2 + c
    seg = w // 2
    half = w % 2
    dx = pltpu.async_copy(x_hbm.at[pl.ds(TPT * w, TPT)], xbuf, s0)
    pltpu.sync_copy(post_hbm.at[:, pl.ds(seg * 128, 128)], pv)
    for j in range(TPT // 16):
        idx0v[pl.ds(16 * j, 16)] = \
            pv[0, pl.ds(half * TPT + 16 * j, 16)].astype(jnp.int32)
        idx1v[pl.ds(16 * j, 16)] = \
            pv[1, pl.ds(half * TPT + 16 * j, 16)].astype(jnp.int32)
    dp0 = pltpu.async_copy(idx0v, p0i_hbm.at[pl.ds(TPT * w, TPT)], s1)
    dx.wait()
    d0 = pltpu.async_copy(xbuf, xs_hbm.at[idx0v], s0)
    dp0.wait()
    dp1 = pltpu.async_copy(idx1v, p1i_hbm.at[pl.ds(TPT * w, TPT)], s1)
    d0.wait()
    d1 = pltpu.async_copy(xbuf, xs_hbm.at[idx1v], s0)
    dp1.wait()
    d1.wait()


def _gout_sc_body(p0_hbm, p1_hbm, ys_hbm, g0_hbm, g1_hbm,
                  p0v, p1v, buf0, buf1, s0, s1, w0, w1):
    c = lax.axis_index("c")
    s = lax.axis_index("s")
    wid = s * 2 + c
    base = wid * TPT
    pltpu.sync_copy(p0_hbm.at[pl.ds(base, TPT)], p0v)
    pltpu.sync_copy(p1_hbm.at[pl.ds(base, TPT)], p1v)
    d0 = pltpu.async_copy(ys_hbm.at[p0v], buf0, s0)
    d1 = pltpu.async_copy(ys_hbm.at[p1v], buf1, s1)
    d0.wait()
    wb0 = pltpu.async_copy(buf0, g0_hbm.at[pl.ds(base, TPT)], w0)
    d1.wait()
    wb1 = pltpu.async_copy(buf1, g1_hbm.at[pl.ds(base, TPT)], w1)
    wb0.wait()
    wb1.wait()


def _experts_body(ce_ref, cv_ref, xs_ref, we1_ref, be1_ref, we2_ref, be2_ref,
                  we3_ref, be3_ref, ys_ref):
    c = pl.program_id(0)

    @pl.when(cv_ref[c] == 1)
    def _compute():
        xb = xs_ref[...]
        h1 = jnp.maximum(
            jnp.dot(xb, we1_ref[0], preferred_element_type=jnp.float32)
            + be1_ref[0], 0.0)
        h2 = jnp.maximum(
            jnp.dot(h1, we2_ref[0], preferred_element_type=jnp.float32)
            + be2_ref[0], 0.0)
        ys_ref[...] = jnp.dot(
            h2, we3_ref[0], preferred_element_type=jnp.float32) + be3_ref[0]


def _combine_body(meta_ref, g0_ref, g1_ref, out_ref):
    lane = lax.broadcasted_iota(jnp.int32, (NB, EP), 1)
    meta = meta_ref[...]
    wa = jnp.sum(jnp.where(lane == 0, meta, 0.0), axis=1, keepdims=True)
    wb = jnp.sum(jnp.where(lane == 1, meta, 0.0), axis=1, keepdims=True)
    out_ref[...] = wa * g0_ref[...] + wb * g1_ref[...]


@jax.jit
def kernel(x, Wr1, br1, Wr2, br2, Wr3, br3, We1, be1, We2, be2, We3, be3):
    wr3p = jnp.pad(Wr3, ((0, 0), (0, EP - E)))
    br3p = jnp.pad(br3, (0, EP - E)).reshape(1, EP)

    probs_full, meta, post = pl.pallas_call(
        _router_body,
        grid=(NTR + 1,),
        in_specs=[
            pl.BlockSpec((NBR, IN), lambda t: (jnp.minimum(t, NTR - 1), 0)),
            pl.BlockSpec((IN, RH), lambda t: (0, 0)),
            pl.BlockSpec((1, RH), lambda t: (0, 0)),
            pl.BlockSpec((RH, RH2), lambda t: (0, 0)),
            pl.BlockSpec((1, RH2), lambda t: (0, 0)),
            pl.BlockSpec((RH2, EP), lambda t: (0, 0)),
            pl.BlockSpec((1, EP), lambda t: (0, 0)),
        ],
        out_specs=[
            pl.BlockSpec((NBR, EP), lambda t: (jnp.minimum(t, NTR - 1), 0)),
            pl.BlockSpec((NBR, EP), lambda t: (jnp.minimum(t, NTR - 1), 0)),
            pl.BlockSpec((8, N), lambda t: (0, 0)),
        ],
        out_shape=[
            jax.ShapeDtypeStruct((N, EP), jnp.float32),
            jax.ShapeDtypeStruct((N, EP), jnp.float32),
            jax.ShapeDtypeStruct((8, N), jnp.float32),
        ],
        scratch_shapes=[pltpu.VMEM((1, EP), jnp.float32),
                        pltpu.VMEM((8, N), jnp.float32)],
        compiler_params=pltpu.CompilerParams(
            dimension_semantics=("arbitrary",)),
    )(x, Wr1, br1.reshape(1, RH), Wr2, br2.reshape(1, RH2), wr3p, br3p)

    mesh = plsc.VectorSubcoreMesh(core_axis_name="c", subcore_axis_name="s")

    chunk_expert = post[4, :CHA].astype(jnp.int32)
    chunk_valid = post[5, :CHA].astype(jnp.int32)

    scatter_sc = functools.partial(
        pl.kernel,
        out_type=[
            jax.ShapeDtypeStruct((APAD, IN), jnp.float32),
            jax.ShapeDtypeStruct((N,), jnp.int32),
            jax.ShapeDtypeStruct((N,), jnp.int32),
        ],
        mesh=mesh,
        compiler_params=pltpu.CompilerParams(needs_layout_passes=False),
        scratch_types=[
            pltpu.VMEM((8, 128), jnp.float32),
            pltpu.VMEM((TPT,), jnp.int32),
            pltpu.VMEM((TPT,), jnp.int32),
            pltpu.VMEM((TPT, IN), jnp.float32),
            pltpu.SemaphoreType.DMA,
            pltpu.SemaphoreType.DMA,
        ],
    )(_scatter_sc_body)
    xs, pos0, pos1 = scatter_sc(post, x)

    ys = pl.pallas_call(
        _experts_body,
        grid_spec=pltpu.PrefetchScalarGridSpec(
            num_scalar_prefetch=2,
            grid=(NCHG,),
            in_specs=[
                pl.BlockSpec((B, IN), lambda c, ce, cv: (c, 0)),
                pl.BlockSpec((1, IN, H1), lambda c, ce, cv: (ce[c], 0, 0)),
                pl.BlockSpec((1, 1, H1), lambda c, ce, cv: (ce[c], 0, 0)),
                pl.BlockSpec((1, H1, H2), lambda c, ce, cv: (ce[c], 0, 0)),
                pl.BlockSpec((1, 1, H2), lambda c, ce, cv: (ce[c], 0, 0)),
                pl.BlockSpec((1, H2, NC), lambda c, ce, cv: (ce[c], 0, 0)),
                pl.BlockSpec((1, 1, NC), lambda c, ce, cv: (ce[c], 0, 0)),
            ],
            out_specs=pl.BlockSpec((B, NC), lambda c, ce, cv: (c, 0)),
        ),
        out_shape=jax.ShapeDtypeStruct((APAD, NC), jnp.float32),
        compiler_params=pltpu.CompilerParams(
            dimension_semantics=("arbitrary",)),
    )(chunk_expert, chunk_valid, xs, We1, be1.reshape(E, 1, H1),
      We2, be2.reshape(E, 1, H2), We3, be3.reshape(E, 1, NC))

    gout_sc = functools.partial(
        pl.kernel,
        out_type=[
            jax.ShapeDtypeStruct((N, NC), jnp.float32),
            jax.ShapeDtypeStruct((N, NC), jnp.float32),
        ],
        mesh=mesh,
        compiler_params=pltpu.CompilerParams(needs_layout_passes=False),
        scratch_types=[
            pltpu.VMEM((TPT,), jnp.int32),
            pltpu.VMEM((TPT,), jnp.int32),
            pltpu.VMEM((TPT, NC), jnp.float32),
            pltpu.VMEM((TPT, NC), jnp.float32),
            pltpu.SemaphoreType.DMA,
            pltpu.SemaphoreType.DMA,
            pltpu.SemaphoreType.DMA,
            pltpu.SemaphoreType.DMA,
        ],
    )(_gout_sc_body)
    g0, g1 = gout_sc(pos0, pos1, ys)

    out = pl.pallas_call(
        _combine_body,
        grid=(NT,),
        in_specs=[
            pl.BlockSpec((NB, EP), lambda t: (t, 0)),
            pl.BlockSpec((NB, NC), lambda t: (t, 0)),
            pl.BlockSpec((NB, NC), lambda t: (t, 0)),
        ],
        out_specs=pl.BlockSpec((NB, NC), lambda t: (t, 0)),
        out_shape=jax.ShapeDtypeStruct((N, NC), jnp.float32),
    )(meta, g0, g1)

    return out, probs_full[:, :E]
